# Initial kernel scaffold; baseline (speedup 1.0000x reference)
import jax
import jax.numpy as jnp
from jax.experimental import pallas as pl


def kernel(x, edge_index, batch, W1, b1, W2, b2, Wfc, bfc):
    raise NotImplementedError("write your pallas kernel here")


# speedup vs baseline: 28.0014x; 28.0014x over previous
import functools

import jax
import jax.numpy as jnp
from jax import lax
from jax.experimental import pallas as pl
from jax.experimental.pallas import tpu as pltpu
from jax.experimental.pallas import tpu_sc as plsc

_N = 10000
_E = 320000
_D = 128
_H = 16
_Y = 64
_G = 64

_NC = 2
_NS = 16
_NW = _NC * _NS
_CH = 128
_CPT = -(-_E // (_NW * _CH))
_EPAD = _NW * _CPT * _CH
_NPAD = 10240
_RPT = _NPAD // _NS
_PADIDX = _N

_mesh = plsc.VectorSubcoreMesh(core_axis_name="c", subcore_axis_name="s")


@functools.partial(
    pl.kernel,
    out_type=jax.ShapeDtypeStruct((_NC, _NPAD, _H), jnp.float32),
    mesh=_mesh,
    scratch_types=[
        pltpu.VMEM((_CPT, _CH), jnp.int32),
        pltpu.VMEM((_CPT, _CH), jnp.int32),
        pltpu.VMEM((_CH, _H), jnp.float32),
        pltpu.VMEM_SHARED((_NPAD, _H), jnp.float32),
        pltpu.SemaphoreType.DMA,
    ],
    compiler_params=pltpu.CompilerParams(use_tc_tiling_on_sc=False),
)
def _sc_pass(zeros_h, table_h, src_h, dst_h, out_h, sidx, didx, buf, acc, sem):
    c = lax.axis_index("c")
    s = lax.axis_index("s")
    w = c * _NS + s
    pltpu.sync_copy(zeros_h.at[pl.ds(s * _RPT, _RPT)],
                    acc.at[pl.ds(s * _RPT, _RPT)])
    pltpu.sync_copy(src_h.at[w], sidx)
    pltpu.sync_copy(dst_h.at[w], didx)
    plsc.subcore_barrier()

    def chunk(j, carry):
        pltpu.async_copy(table_h.at[sidx.at[j]], buf, sem).wait()
        pltpu.sync_copy(buf, acc.at[didx.at[j]], add=True)
        return carry

    lax.fori_loop(0, _CPT, chunk, 0)
    plsc.subcore_barrier()
    pltpu.sync_copy(acc.at[pl.ds(s * _RPT, _RPT)],
                    out_h.at[c, pl.ds(s * _RPT, _RPT)])


def _tc_pre(x_ref, w1_ref, degp_ref, y1_ref, dinv_ref):
    deg = degp_ref[0, :, 0:1] + degp_ref[1, :, 0:1] + 1.0
    dinv = lax.rsqrt(deg)
    xw = jnp.dot(x_ref[...], w1_ref[...], preferred_element_type=jnp.float32)
    y1_ref[...] = dinv * xw
    dinv_ref[...] = dinv


def _tc_mid(z_ref, y1_ref, dinv_ref, b1_ref, y2_ref):
    dinv = dinv_ref[...]
    z = z_ref[0] + z_ref[1] + y1_ref[...]
    h1 = jnp.maximum(dinv * z + b1_ref[...], 0.0)
    y2_ref[...] = dinv * h1


def _tc_post(z_ref, y2_ref, dinv_ref, b2_ref, w2_ref, wfc_ref, bfc_ref,
             batch_ref, out_ref):
    a = dinv_ref[...] * (z_ref[0] + z_ref[1] + y2_ref[...])
    h2 = jnp.maximum(
        jnp.dot(a, w2_ref[...], preferred_element_type=jnp.float32)
        + b2_ref[...], 0.0)
    s = jnp.dot(h2, wfc_ref[...], preferred_element_type=jnp.float32)
    onehot = (lax.broadcasted_iota(jnp.int32, (_G, _NPAD), 0)
              == batch_ref[...]).astype(jnp.float32)
    pooled = jnp.dot(onehot, s, preferred_element_type=jnp.float32)
    logit = pooled + bfc_ref[...]
    out_ref[...] = 1.0 / (1.0 + jnp.exp(-logit))


def kernel(x, edge_index, batch, W1, b1, W2, b2, Wfc, bfc):
    f32 = jnp.float32
    src = edge_index[0].astype(jnp.int32)
    dst = edge_index[1].astype(jnp.int32)
    pad = jnp.full((_EPAD - _E,), _PADIDX, jnp.int32)
    src_p = jnp.concatenate([src, pad]).reshape(_NW, _CPT, _CH)
    dst_p = jnp.concatenate([dst, pad]).reshape(_NW, _CPT, _CH)
    x_p = jnp.pad(x, ((0, _NPAD - _N), (0, 0)))
    batch_p = jnp.pad(batch.astype(jnp.int32), (0, _NPAD - _N),
                      constant_values=_G).reshape(1, _NPAD)
    zeros16 = jnp.zeros((_NPAD, _H), f32)
    ones16 = jnp.ones((_NPAD, _H), f32)

    degp = _sc_pass(zeros16, ones16, dst_p, dst_p)

    y1, dinv = pl.pallas_call(
        _tc_pre,
        out_shape=[jax.ShapeDtypeStruct((_NPAD, _H), f32),
                   jax.ShapeDtypeStruct((_NPAD, 1), f32)],
    )(x_p, W1, degp)

    z1 = _sc_pass(zeros16, y1, src_p, dst_p)

    y2 = pl.pallas_call(
        _tc_mid,
        out_shape=jax.ShapeDtypeStruct((_NPAD, _H), f32),
    )(z1, y1, dinv, b1.reshape(1, _H))

    z2 = _sc_pass(zeros16, y2, src_p, dst_p)

    out = pl.pallas_call(
        _tc_post,
        out_shape=jax.ShapeDtypeStruct((_G, 1), f32),
    )(z2, y2, dinv, b2.reshape(1, _Y), W2, Wfc, bfc.reshape(1, 1), batch_p)

    return out


# speedup vs baseline: 40.5649x; 1.4487x over previous
import functools

import jax
import jax.numpy as jnp
from jax import lax
from jax.experimental import pallas as pl
from jax.experimental.pallas import tpu as pltpu
from jax.experimental.pallas import tpu_sc as plsc

_N = 10000
_E = 320000
_D = 128
_H = 16
_Y = 64
_G = 64

_NC = 2
_NS = 16
_NW = _NC * _NS
_CH = 128
_CPT = 80
_EPAD = _NW * _CPT * _CH
_NPAD = 10240
_RPT = _NPAD // _NS
_PADIDX = _N

_mesh = plsc.VectorSubcoreMesh(core_axis_name="c", subcore_axis_name="s")


@functools.partial(
    pl.kernel,
    out_type=jax.ShapeDtypeStruct((_NW, _NPAD), jnp.float32),
    mesh=_mesh,
    scratch_types=[
        pltpu.VMEM((_CPT, _CH), jnp.int32),
        pltpu.VMEM((_NPAD,), jnp.float32),
    ],
    compiler_params=pltpu.CompilerParams(use_tc_tiling_on_sc=False,
                                        needs_layout_passes=False),
)
def _sc_deg(dst_h, out_h, didx, hist):
    c = lax.axis_index("c")
    s = lax.axis_index("s")
    w = c * _NS + s
    pltpu.sync_copy(dst_h.at[w], didx)

    def zero(i, carry):
        hist[pl.ds(pl.multiple_of(i * 16, 16), 16)] = jnp.zeros((16,),
                                                                jnp.float32)
        return carry

    lax.fori_loop(0, _NPAD // 16, zero, 0)

    ones = jnp.ones((16,), jnp.float32)

    def chunk(j, carry):
        for k in range(_CH // 16):
            idx = didx[j, pl.ds(k * 16, 16)]
            plsc.addupdate_scatter(hist, [idx], ones)
        return carry

    lax.fori_loop(0, _CPT, chunk, 0)
    pltpu.sync_copy(hist, out_h.at[w])


@functools.partial(
    pl.kernel,
    out_type=jax.ShapeDtypeStruct((_NC, _NPAD, _H), jnp.float32),
    mesh=_mesh,
    scratch_types=[
        pltpu.VMEM((_CPT, _CH), jnp.int32),
        pltpu.VMEM((_CPT, _CH), jnp.int32),
        pltpu.VMEM((_CH, _H), jnp.float32),
        pltpu.VMEM((_CH, _H), jnp.float32),
        pltpu.VMEM_SHARED((_NPAD, _H), jnp.float32),
        pltpu.SemaphoreType.DMA,
        pltpu.SemaphoreType.DMA,
    ],
    compiler_params=pltpu.CompilerParams(use_tc_tiling_on_sc=False,
                                        needs_layout_passes=False),
)
def _sc_mp(zeros_h, table_h, src_h, dst_h, out_h,
           sidx, didx, buf_a, buf_b, acc, sem_a, sem_b):
    c = lax.axis_index("c")
    s = lax.axis_index("s")
    w = c * _NS + s
    pltpu.sync_copy(zeros_h.at[pl.ds(s * _RPT, _RPT)],
                    acc.at[pl.ds(s * _RPT, _RPT)])
    pltpu.sync_copy(src_h.at[w], sidx)
    pltpu.sync_copy(dst_h.at[w], didx)
    plsc.subcore_barrier()

    pltpu.async_copy(table_h.at[sidx.at[0]], buf_a, sem_a)

    def step(gg, carry):
        ja = 2 * gg
        jb = 2 * gg + 1
        pltpu.async_copy(table_h.at[sidx.at[jb]], buf_b, sem_b)
        pltpu.make_async_copy(table_h.at[sidx.at[ja]], buf_a, sem_a).wait()
        pltpu.sync_copy(buf_a, acc.at[didx.at[ja]], add=True)

        @pl.when(gg + 1 < _CPT // 2)
        def _():
            pltpu.async_copy(table_h.at[sidx.at[ja + 2]], buf_a, sem_a)

        pltpu.make_async_copy(table_h.at[sidx.at[jb]], buf_b, sem_b).wait()
        pltpu.sync_copy(buf_b, acc.at[didx.at[jb]], add=True)
        return carry

    lax.fori_loop(0, _CPT // 2, step, 0)
    plsc.subcore_barrier()
    pltpu.sync_copy(acc.at[pl.ds(s * _RPT, _RPT)],
                    out_h.at[c, pl.ds(s * _RPT, _RPT)])


def _tc_pre(x_ref, w1_ref, degp_ref, onesw_ref, y1_ref, dinv_ref):
    deg = lax.dot_general(degp_ref[...], onesw_ref[...],
                          (((0,), (0,)), ((), ())),
                          preferred_element_type=jnp.float32) + 1.0
    dinv = lax.rsqrt(deg)
    xw = jnp.dot(x_ref[...], w1_ref[...], preferred_element_type=jnp.float32)
    y1_ref[...] = dinv * xw
    dinv_ref[...] = dinv


def _tc_mid(z_ref, y1_ref, dinv_ref, b1_ref, y2_ref):
    dinv = dinv_ref[...]
    z = z_ref[0] + z_ref[1] + y1_ref[...]
    h1 = jnp.maximum(dinv * z + b1_ref[...], 0.0)
    y2_ref[...] = dinv * h1


def _tc_post(z_ref, y2_ref, dinv_ref, b2_ref, w2_ref, wfc_ref, bfc_ref,
             batch_ref, out_ref):
    a = dinv_ref[...] * (z_ref[0] + z_ref[1] + y2_ref[...])
    h2 = jnp.maximum(
        jnp.dot(a, w2_ref[...], preferred_element_type=jnp.float32)
        + b2_ref[...], 0.0)
    s = jnp.dot(h2, wfc_ref[...], preferred_element_type=jnp.float32)
    onehot = (lax.broadcasted_iota(jnp.int32, (_G, _NPAD), 0)
              == batch_ref[...]).astype(jnp.float32)
    pooled = jnp.dot(onehot, s, preferred_element_type=jnp.float32)
    logit = pooled + bfc_ref[...]
    out_ref[...] = 1.0 / (1.0 + jnp.exp(-logit))


def kernel(x, edge_index, batch, W1, b1, W2, b2, Wfc, bfc):
    f32 = jnp.float32
    src = edge_index[0].astype(jnp.int32)
    dst = edge_index[1].astype(jnp.int32)
    pad = jnp.full((_EPAD - _E,), _PADIDX, jnp.int32)
    src_p = jnp.concatenate([src, pad]).reshape(_NW, _CPT, _CH)
    dst_p = jnp.concatenate([dst, pad]).reshape(_NW, _CPT, _CH)
    x_p = jnp.pad(x, ((0, _NPAD - _N), (0, 0)))
    batch_p = jnp.pad(batch.astype(jnp.int32), (0, _NPAD - _N),
                      constant_values=_G).reshape(1, _NPAD)
    zeros16 = jnp.zeros((_NPAD, _H), f32)
    onesw = jnp.ones((_NW, 1), f32)

    degp = _sc_deg(dst_p)

    y1, dinv = pl.pallas_call(
        _tc_pre,
        out_shape=[jax.ShapeDtypeStruct((_NPAD, _H), f32),
                   jax.ShapeDtypeStruct((_NPAD, 1), f32)],
    )(x_p, W1, degp, onesw)

    z1 = _sc_mp(zeros16, y1, src_p, dst_p)

    y2 = pl.pallas_call(
        _tc_mid,
        out_shape=jax.ShapeDtypeStruct((_NPAD, _H), f32),
    )(z1, y1, dinv, b1.reshape(1, _H))

    z2 = _sc_mp(zeros16, y2, src_p, dst_p)

    out = pl.pallas_call(
        _tc_post,
        out_shape=jax.ShapeDtypeStruct((_G, 1), f32),
    )(z2, y2, dinv, b2.reshape(1, _Y), W2, Wfc, bfc.reshape(1, 1), batch_p)

    return out


# speedup vs baseline: 41.5371x; 1.0240x over previous
import functools

import jax
import jax.numpy as jnp
from jax import lax
from jax.experimental import pallas as pl
from jax.experimental.pallas import tpu as pltpu
from jax.experimental.pallas import tpu_sc as plsc

_N = 10000
_E = 320000
_D = 128
_H = 16
_Y = 64
_G = 64

_NC = 2
_NS = 16
_NW = _NC * _NS
_CH = 128
_CPT = 80
_EPAD = _NW * _CPT * _CH
_NPAD = 10240
_RPT = _NPAD // _NS
_PADIDX = _N

_mesh = plsc.VectorSubcoreMesh(core_axis_name="c", subcore_axis_name="s")
_params = pltpu.CompilerParams(use_tc_tiling_on_sc=False,
                               needs_layout_passes=False)


def _lane_bcast(v, k):
    idx = jnp.full((16, 1), k, dtype=jnp.int32)
    dn = lax.GatherDimensionNumbers(offset_dims=(), collapsed_slice_dims=(0,),
                                    start_index_map=(0,))
    return lax.gather(v, idx, dn, (1,),
                      mode=lax.GatherScatterMode.PROMISE_IN_BOUNDS)


def _rsqrt16(x):
    i = plsc.bitcast(x, jnp.int32)
    i = jnp.int32(0x5F3759DF) - lax.shift_right_arithmetic(i, 1)
    y = plsc.bitcast(i, jnp.float32)
    for _ in range(3):
        y = y * (1.5 - 0.5 * x * y * y)
    return y


@functools.partial(
    pl.kernel,
    out_type=jax.ShapeDtypeStruct((_NW, _NPAD), jnp.float32),
    mesh=_mesh,
    scratch_types=[
        pltpu.VMEM((_CPT, _CH), jnp.int32),
        pltpu.VMEM((_NPAD,), jnp.float32),
    ],
    compiler_params=_params,
)
def _sc_deg(dst_h, out_h, didx, hist):
    c = lax.axis_index("c")
    s = lax.axis_index("s")
    w = c * _NS + s
    pltpu.sync_copy(dst_h.at[w], didx)

    def zero(i, carry):
        hist[pl.ds(pl.multiple_of(i * 16, 16), 16)] = jnp.zeros((16,),
                                                                jnp.float32)
        return carry

    lax.fori_loop(0, _NPAD // 16, zero, 0)

    ones = jnp.ones((16,), jnp.float32)

    def chunk(j, carry):
        for k in range(_CH // 16):
            idx = didx[j, pl.ds(k * 16, 16)]
            plsc.addupdate_scatter(hist, [idx], ones)
        return carry

    lax.fori_loop(0, _CPT, chunk, 0)
    pltpu.sync_copy(hist, out_h.at[w])


def _sc_mp_body(layer2, degp_h, xw1_h, z1p_h, b1_h, src_h, dst_h, zeros_h,
                zout_h, ytab_h,
                sidx, didx, buf_a, buf_b, dbuf, ybuf, zbuf, bvec, acc,
                sem_a, sem_b):
    c = lax.axis_index("c")
    s = lax.axis_index("s")
    w = c * _NS + s
    base = s * _RPT
    pltpu.sync_copy(zeros_h.at[pl.ds(base, _RPT)], acc.at[pl.ds(base, _RPT)])
    pltpu.sync_copy(src_h.at[w], sidx)
    pltpu.sync_copy(dst_h.at[w], didx)

    pltpu.sync_copy(degp_h.at[:, pl.ds(base, _RPT)], dbuf)
    pltpu.sync_copy(xw1_h.at[pl.ds(base, _RPT)], ybuf)
    if layer2:
        pltpu.sync_copy(z1p_h.at[0, pl.ds(base, _RPT)], zbuf.at[0])
        pltpu.sync_copy(z1p_h.at[1, pl.ds(base, _RPT)], zbuf.at[1])
        pltpu.sync_copy(b1_h, bvec)

    def grp(g, carry):
        o = pl.multiple_of(g * 16, 16)
        d = dbuf[0, pl.ds(o, 16)]
        for r in range(1, _NW):
            d = d + dbuf[r, pl.ds(o, 16)]
        dv = _rsqrt16(d + 1.0)
        for k in range(16):
            dvk = _lane_bcast(dv, k)
            if layer2:
                y1 = dvk * ybuf[o + k, :]
                z = zbuf[0, o + k, :] + zbuf[1, o + k, :] + y1
                h1 = jnp.maximum(dvk * z + bvec[:], 0.0)
                ybuf[o + k, :] = dvk * h1
            else:
                ybuf[o + k, :] = dvk * ybuf[o + k, :]
        return carry

    lax.fori_loop(0, _RPT // 16, grp, 0)
    pltpu.sync_copy(ybuf, ytab_h.at[pl.ds(c * _NPAD + base, _RPT)])
    plsc.subcore_barrier()

    pltpu.async_copy(ytab_h.at[sidx.at[0]], buf_a, sem_a)

    def step(gg, carry):
        ja = 2 * gg
        jb = 2 * gg + 1
        pltpu.async_copy(ytab_h.at[sidx.at[jb]], buf_b, sem_b)
        pltpu.make_async_copy(ytab_h.at[sidx.at[ja]], buf_a, sem_a).wait()
        pltpu.sync_copy(buf_a, acc.at[didx.at[ja]], add=True)

        @pl.when(gg + 1 < _CPT // 2)
        def _():
            pltpu.async_copy(ytab_h.at[sidx.at[ja + 2]], buf_a, sem_a)

        pltpu.make_async_copy(ytab_h.at[sidx.at[jb]], buf_b, sem_b).wait()
        pltpu.sync_copy(buf_b, acc.at[didx.at[jb]], add=True)
        return carry

    lax.fori_loop(0, _CPT // 2, step, 0)
    plsc.subcore_barrier()
    pltpu.sync_copy(acc.at[pl.ds(base, _RPT)],
                    zout_h.at[c, pl.ds(base, _RPT)])


_mp_out = [jax.ShapeDtypeStruct((_NC, _NPAD, _H), jnp.float32),
           jax.ShapeDtypeStruct((_NC * _NPAD, _H), jnp.float32)]
_mp_scratch = [
    pltpu.VMEM((_CPT, _CH), jnp.int32),
    pltpu.VMEM((_CPT, _CH), jnp.int32),
    pltpu.VMEM((_CH, _H), jnp.float32),
    pltpu.VMEM((_CH, _H), jnp.float32),
    pltpu.VMEM((_NW, _RPT), jnp.float32),
    pltpu.VMEM((_RPT, _H), jnp.float32),
    pltpu.VMEM((2, _RPT, _H), jnp.float32),
    pltpu.VMEM((_H,), jnp.float32),
    pltpu.VMEM_SHARED((_NPAD, _H), jnp.float32),
    pltpu.SemaphoreType.DMA,
    pltpu.SemaphoreType.DMA,
]

_sc_mp1 = functools.partial(
    pl.kernel, out_type=_mp_out, mesh=_mesh, scratch_types=_mp_scratch,
    compiler_params=_params)(functools.partial(_sc_mp_body, False))

_sc_mp2 = functools.partial(
    pl.kernel, out_type=_mp_out, mesh=_mesh, scratch_types=_mp_scratch,
    compiler_params=_params)(functools.partial(_sc_mp_body, True))


def _tc_mm1(x_ref, w1_ref, xw_ref):
    xw_ref[...] = jnp.dot(x_ref[...], w1_ref[...],
                          preferred_element_type=jnp.float32)


def _tc_post(degp_ref, onesw_ref, xw1_ref, z1_ref, z2_ref, b1_ref, b2_ref,
             w2_ref, wfc_ref, bfc_ref, batch_ref, out_ref):
    deg = lax.dot_general(degp_ref[...], onesw_ref[...],
                          (((0,), (0,)), ((), ())),
                          preferred_element_type=jnp.float32) + 1.0
    dinv = lax.rsqrt(deg)
    y1 = dinv * xw1_ref[...]
    h1 = jnp.maximum(dinv * (z1_ref[0] + z1_ref[1] + y1) + b1_ref[...], 0.0)
    y2 = dinv * h1
    a = dinv * (z2_ref[0] + z2_ref[1] + y2)
    h2 = jnp.maximum(
        jnp.dot(a, w2_ref[...], preferred_element_type=jnp.float32)
        + b2_ref[...], 0.0)
    s = jnp.dot(h2, wfc_ref[...], preferred_element_type=jnp.float32)
    onehot = (lax.broadcasted_iota(jnp.int32, (_G, _NPAD), 0)
              == batch_ref[...]).astype(jnp.float32)
    pooled = jnp.dot(onehot, s, preferred_element_type=jnp.float32)
    logit = pooled + bfc_ref[...]
    out_ref[...] = 1.0 / (1.0 + jnp.exp(-logit))


def kernel(x, edge_index, batch, W1, b1, W2, b2, Wfc, bfc):
    f32 = jnp.float32
    src = edge_index[0].astype(jnp.int32)
    dst = edge_index[1].astype(jnp.int32)
    pad = jnp.full((_EPAD - _E,), _PADIDX, jnp.int32)
    coff = (jnp.arange(_NW, dtype=jnp.int32) // _NS * _NPAD)[:, None, None]
    src_p = jnp.concatenate([src, pad]).reshape(_NW, _CPT, _CH) + coff
    dst_p = jnp.concatenate([dst, pad]).reshape(_NW, _CPT, _CH)
    x_p = jnp.pad(x, ((0, _NPAD - _N), (0, 0)))
    batch_p = jnp.pad(batch.astype(jnp.int32), (0, _NPAD - _N),
                      constant_values=_G).reshape(1, _NPAD)
    zeros16 = jnp.zeros((_NPAD, _H), f32)
    onesw = jnp.ones((_NW, 1), f32)
    b1r = b1.astype(f32)

    degp = _sc_deg(dst_p)

    xw1 = pl.pallas_call(
        _tc_mm1, out_shape=jax.ShapeDtypeStruct((_NPAD, _H), f32),
    )(x_p, W1)

    zdummy = jnp.zeros((_NC, _NPAD, _H), f32)
    z1, _ = _sc_mp1(degp, xw1, zdummy, b1r, src_p, dst_p, zeros16)
    z2, _ = _sc_mp2(degp, xw1, z1, b1r, src_p, dst_p, zeros16)

    out = pl.pallas_call(
        _tc_post, out_shape=jax.ShapeDtypeStruct((_G, 1), f32),
    )(degp, onesw, xw1, z1, z2, b1.reshape(1, _H), b2.reshape(1, _Y),
      W2, Wfc, bfc.reshape(1, 1), batch_p)

    return out


# speedup vs baseline: 54.3454x; 1.3084x over previous
import functools

import jax
import jax.numpy as jnp
from jax import lax
from jax.experimental import pallas as pl
from jax.experimental.pallas import tpu as pltpu
from jax.experimental.pallas import tpu_sc as plsc

_N = 10000
_E = 320000
_D = 128
_H = 16
_Y = 64
_G = 64

_NC = 2
_NS = 16
_NW = _NC * _NS
_CH = 128
_CPT = 80
_EPAD = _NW * _CPT * _CH
_NPAD = 10240
_RPT = _NPAD // _NS
_PADIDX = _N

_mesh = plsc.VectorSubcoreMesh(core_axis_name="c", subcore_axis_name="s")
_params = pltpu.CompilerParams(use_tc_tiling_on_sc=False,
                               needs_layout_passes=False)


def _lane_bcast(v, k):
    idx = jnp.full((16, 1), k, dtype=jnp.int32)
    dn = lax.GatherDimensionNumbers(offset_dims=(), collapsed_slice_dims=(0,),
                                    start_index_map=(0,))
    return lax.gather(v, idx, dn, (1,),
                      mode=lax.GatherScatterMode.PROMISE_IN_BOUNDS)


def _rsqrt16(x):
    i = plsc.bitcast(x, jnp.int32)
    i = jnp.int32(0x5F3759DF) - lax.shift_right_arithmetic(i, 1)
    y = plsc.bitcast(i, jnp.float32)
    for _ in range(3):
        y = y * (1.5 - 0.5 * x * y * y)
    return y


@functools.partial(
    pl.kernel,
    out_type=jax.ShapeDtypeStruct((_NW, _NPAD), jnp.float32),
    mesh=_mesh,
    scratch_types=[
        pltpu.VMEM((_CPT, _CH), jnp.int32),
        pltpu.VMEM((_NPAD,), jnp.float32),
    ],
    compiler_params=_params,
)
def _sc_deg(dst_h, out_h, didx, hist):
    c = lax.axis_index("c")
    s = lax.axis_index("s")
    w = c * _NS + s
    pltpu.sync_copy(dst_h.at[w], didx)

    def zero(i, carry):
        hist[pl.ds(pl.multiple_of(i * 16, 16), 16)] = jnp.zeros((16,),
                                                                jnp.float32)
        return carry

    lax.fori_loop(0, _NPAD // 16, zero, 0)

    ones = jnp.ones((16,), jnp.float32)

    def chunk(j, carry):
        for k in range(_CH // 16):
            idx = didx[j, pl.ds(k * 16, 16)]
            plsc.addupdate_scatter(hist, [idx], ones)
        return carry

    lax.fori_loop(0, _CPT, chunk, 0)
    pltpu.sync_copy(hist, out_h.at[w])


def _sc_mp_body(layer2, *refs):
    if layer2:
        (degp_h, xw1_h, z1p_h, b1_h, src_h, dst_h, zeros_h, zout_h, ytab_h,
         sidx, didx, buf_a, buf_b, dbuf, ybuf, zbuf, bvec, acc,
         sem_a, sem_b) = refs
    else:
        (degp_h, xw1_h, src_h, dst_h, zeros_h, zout_h, ytab_h,
         sidx, didx, buf_a, buf_b, dbuf, ybuf, acc, sem_a, sem_b) = refs
    c = lax.axis_index("c")
    s = lax.axis_index("s")
    w = c * _NS + s
    base = s * _RPT
    pltpu.sync_copy(zeros_h.at[pl.ds(base, _RPT)], acc.at[pl.ds(base, _RPT)])
    pltpu.sync_copy(src_h.at[w], sidx)
    pltpu.sync_copy(dst_h.at[w], didx)

    pltpu.sync_copy(degp_h.at[:, pl.ds(base, _RPT)], dbuf)
    pltpu.sync_copy(xw1_h.at[pl.ds(base, _RPT)], ybuf)
    if layer2:
        pltpu.sync_copy(z1p_h.at[0, pl.ds(base, _RPT)], zbuf.at[0])
        pltpu.sync_copy(z1p_h.at[1, pl.ds(base, _RPT)], zbuf.at[1])
        pltpu.sync_copy(b1_h, bvec)

    def grp(g, carry):
        o = pl.multiple_of(g * 16, 16)
        d = dbuf[0, pl.ds(o, 16)]
        for r in range(1, _NW):
            d = d + dbuf[r, pl.ds(o, 16)]
        dv = _rsqrt16(d + 1.0)
        for k in range(16):
            dvk = _lane_bcast(dv, k)
            if layer2:
                y1 = dvk * ybuf[o + k, :]
                z = zbuf[0, o + k, :] + zbuf[1, o + k, :] + y1
                h1 = jnp.maximum(dvk * z + bvec[:], 0.0)
                ybuf[o + k, :] = dvk * h1
            else:
                ybuf[o + k, :] = dvk * ybuf[o + k, :]
        return carry

    lax.fori_loop(0, _RPT // 16, grp, 0)
    pltpu.sync_copy(ybuf, ytab_h.at[pl.ds(c * _NPAD + base, _RPT)])
    plsc.subcore_barrier()

    pltpu.async_copy(ytab_h.at[sidx.at[0]], buf_a, sem_a)

    def step(gg, carry):
        ja = 2 * gg
        jb = 2 * gg + 1
        pltpu.async_copy(ytab_h.at[sidx.at[jb]], buf_b, sem_b)
        pltpu.make_async_copy(ytab_h.at[sidx.at[ja]], buf_a, sem_a).wait()
        pltpu.sync_copy(buf_a, acc.at[didx.at[ja]], add=True)

        @pl.when(gg + 1 < _CPT // 2)
        def _():
            pltpu.async_copy(ytab_h.at[sidx.at[ja + 2]], buf_a, sem_a)

        pltpu.make_async_copy(ytab_h.at[sidx.at[jb]], buf_b, sem_b).wait()
        pltpu.sync_copy(buf_b, acc.at[didx.at[jb]], add=True)
        return carry

    lax.fori_loop(0, _CPT // 2, step, 0)
    plsc.subcore_barrier()
    pltpu.sync_copy(acc.at[pl.ds(base, _RPT)],
                    zout_h.at[c, pl.ds(base, _RPT)])


_mp_out = [jax.ShapeDtypeStruct((_NC, _NPAD, _H), jnp.float32),
           jax.ShapeDtypeStruct((_NC * _NPAD, _H), jnp.float32)]
_mp_scratch1 = [
    pltpu.VMEM((_CPT, _CH), jnp.int32),
    pltpu.VMEM((_CPT, _CH), jnp.int32),
    pltpu.VMEM((_CH, _H), jnp.float32),
    pltpu.VMEM((_CH, _H), jnp.float32),
    pltpu.VMEM((_NW, _RPT), jnp.float32),
    pltpu.VMEM((_RPT, _H), jnp.float32),
    pltpu.VMEM_SHARED((_NPAD, _H), jnp.float32),
    pltpu.SemaphoreType.DMA,
    pltpu.SemaphoreType.DMA,
]
_mp_scratch2 = _mp_scratch1[:6] + [
    pltpu.VMEM((2, _RPT, _H), jnp.float32),
    pltpu.VMEM((_H,), jnp.float32),
] + _mp_scratch1[6:]

_sc_mp1 = functools.partial(
    pl.kernel, out_type=_mp_out, mesh=_mesh, scratch_types=_mp_scratch1,
    compiler_params=_params)(functools.partial(_sc_mp_body, False))

_sc_mp2 = functools.partial(
    pl.kernel, out_type=_mp_out, mesh=_mesh, scratch_types=_mp_scratch2,
    compiler_params=_params)(functools.partial(_sc_mp_body, True))


def _tc_mm1(x_ref, w1_ref, xw_ref):
    xw_ref[...] = jnp.dot(x_ref[...], w1_ref[...],
                          preferred_element_type=jnp.float32)


def _tc_post(degp_ref, onesw_ref, xw1_ref, z1_ref, z2_ref, b1_ref, b2_ref,
             w2_ref, wfc_ref, bfc_ref, batch_ref, out_ref):
    deg = lax.dot_general(degp_ref[...], onesw_ref[...],
                          (((0,), (0,)), ((), ())),
                          preferred_element_type=jnp.float32) + 1.0
    dinv = lax.rsqrt(deg)
    y1 = dinv * xw1_ref[...]
    h1 = jnp.maximum(dinv * (z1_ref[0] + z1_ref[1] + y1) + b1_ref[...], 0.0)
    y2 = dinv * h1
    a = dinv * (z2_ref[0] + z2_ref[1] + y2)
    h2 = jnp.maximum(
        jnp.dot(a, w2_ref[...], preferred_element_type=jnp.float32)
        + b2_ref[...], 0.0)
    s = jnp.dot(h2, wfc_ref[...], preferred_element_type=jnp.float32)
    onehot = (lax.broadcasted_iota(jnp.int32, (_G, _NPAD), 0)
              == batch_ref[...]).astype(jnp.float32)
    pooled = jnp.dot(onehot, s, preferred_element_type=jnp.float32)
    logit = pooled + bfc_ref[...]
    out_ref[...] = 1.0 / (1.0 + jnp.exp(-logit))


def kernel(x, edge_index, batch, W1, b1, W2, b2, Wfc, bfc):
    f32 = jnp.float32
    src = edge_index[0].astype(jnp.int32)
    dst = edge_index[1].astype(jnp.int32)
    pad = _PADIDX + jnp.arange(_EPAD - _E, dtype=jnp.int32) % (_NPAD - _N)
    coff = (jnp.arange(_NW, dtype=jnp.int32) // _NS * _NPAD)[:, None, None]
    src_p = jnp.concatenate([src, pad]).reshape(_NW, _CPT, _CH) + coff
    dst_p = jnp.concatenate([dst, pad]).reshape(_NW, _CPT, _CH)
    x_p = jnp.pad(x, ((0, _NPAD - _N), (0, 0)))
    batch_p = jnp.pad(batch.astype(jnp.int32), (0, _NPAD - _N),
                      constant_values=_G).reshape(1, _NPAD)
    zeros16 = jnp.zeros((_NPAD, _H), f32)
    onesw = jnp.ones((_NW, 1), f32)
    b1r = b1.astype(f32)

    degp = _sc_deg(dst_p)

    xw1 = pl.pallas_call(
        _tc_mm1, out_shape=jax.ShapeDtypeStruct((_NPAD, _H), f32),
    )(x_p, W1)

    z1, _ = _sc_mp1(degp, xw1, src_p, dst_p, zeros16)
    z2, _ = _sc_mp2(degp, xw1, z1, b1r, src_p, dst_p, zeros16)

    out = pl.pallas_call(
        _tc_post, out_shape=jax.ShapeDtypeStruct((_G, 1), f32),
    )(degp, onesw, xw1, z1, z2, b1.reshape(1, _H), b2.reshape(1, _Y),
      W2, Wfc, bfc.reshape(1, 1), batch_p)

    return out


# speedup vs baseline: 57.1946x; 1.0524x over previous
import functools

import jax
import jax.numpy as jnp
from jax import lax
from jax.experimental import pallas as pl
from jax.experimental.pallas import tpu as pltpu
from jax.experimental.pallas import tpu_sc as plsc

_N = 10000
_E = 320000
_D = 128
_H = 16
_Y = 64
_G = 64

_NC = 2
_NS = 16
_NW = _NC * _NS
_CH = 128
_CPT = 80
_EPAD = _NW * _CPT * _CH
_NPAD = 10240
_RPT = _NPAD // _NS
_PADIDX = _N

_mesh = plsc.VectorSubcoreMesh(core_axis_name="c", subcore_axis_name="s")
_params = pltpu.CompilerParams(use_tc_tiling_on_sc=False,
                               needs_layout_passes=False)


def _lane_bcast(v, k):
    idx = jnp.full((16, 1), k, dtype=jnp.int32)
    dn = lax.GatherDimensionNumbers(offset_dims=(), collapsed_slice_dims=(0,),
                                    start_index_map=(0,))
    return lax.gather(v, idx, dn, (1,),
                      mode=lax.GatherScatterMode.PROMISE_IN_BOUNDS)


def _rsqrt16(x):
    i = plsc.bitcast(x, jnp.int32)
    i = jnp.int32(0x5F3759DF) - lax.shift_right_arithmetic(i, 1)
    y = plsc.bitcast(i, jnp.float32)
    for _ in range(3):
        y = y * (1.5 - 0.5 * x * y * y)
    return y


@functools.partial(
    pl.kernel,
    out_type=jax.ShapeDtypeStruct((_NW, _NPAD), jnp.float32),
    mesh=_mesh,
    scratch_types=[
        pltpu.VMEM((_CPT, _CH), jnp.int32),
        pltpu.VMEM((_NPAD,), jnp.float32),
    ],
    compiler_params=_params,
)
def _sc_deg(dst_h, out_h, didx, hist):
    c = lax.axis_index("c")
    s = lax.axis_index("s")
    w = c * _NS + s
    pltpu.sync_copy(dst_h.at[w], didx)

    def zero(i, carry):
        hist[pl.ds(pl.multiple_of(i * 16, 16), 16)] = jnp.zeros((16,),
                                                                jnp.float32)
        return carry

    lax.fori_loop(0, _NPAD // 16, zero, 0)

    ones = jnp.ones((16,), jnp.float32)

    def chunk(j, carry):
        for k in range(_CH // 16):
            idx = didx[j, pl.ds(k * 16, 16)]
            plsc.addupdate_scatter(hist, [idx], ones)
        return carry

    lax.fori_loop(0, _CPT, chunk, 0)
    pltpu.sync_copy(hist, out_h.at[w])


def _sc_mp_body(layer2, *refs):
    if layer2:
        (degp_h, xw1_h, z1p_h, b1_h, src_h, dst_h, zeros_h, zout_h, ytab_h,
         sidx, didx, bufs, dbuf, ybuf, zbuf, bvec, acc,
         g0, g1, g2, g3, s0, s1, s2, s3) = refs
    else:
        (degp_h, xw1_h, src_h, dst_h, zeros_h, zout_h, ytab_h,
         sidx, didx, bufs, dbuf, ybuf, acc,
         g0, g1, g2, g3, s0, s1, s2, s3) = refs
    gsem = [g0, g1, g2, g3]
    ssem = [s0, s1, s2, s3]
    c = lax.axis_index("c")
    s = lax.axis_index("s")
    w = c * _NS + s
    base = s * _RPT
    pltpu.sync_copy(zeros_h.at[pl.ds(base, _RPT)], acc.at[pl.ds(base, _RPT)])
    pltpu.sync_copy(src_h.at[w], sidx)
    pltpu.sync_copy(dst_h.at[w], didx)

    pltpu.sync_copy(degp_h.at[:, pl.ds(base, _RPT)], dbuf)
    pltpu.sync_copy(xw1_h.at[pl.ds(base, _RPT)], ybuf)
    if layer2:
        pltpu.sync_copy(z1p_h.at[0, pl.ds(base, _RPT)], zbuf.at[0])
        pltpu.sync_copy(z1p_h.at[1, pl.ds(base, _RPT)], zbuf.at[1])
        pltpu.sync_copy(b1_h, bvec)

    def grp(g, carry):
        o = pl.multiple_of(g * 16, 16)
        d = dbuf[0, pl.ds(o, 16)]
        for r in range(1, _NW):
            d = d + dbuf[r, pl.ds(o, 16)]
        dv = _rsqrt16(d + 1.0)
        for k in range(16):
            dvk = _lane_bcast(dv, k)
            if layer2:
                y1 = dvk * ybuf[o + k, :]
                z = zbuf[0, o + k, :] + zbuf[1, o + k, :] + y1
                h1 = jnp.maximum(dvk * z + bvec[:], 0.0)
                ybuf[o + k, :] = dvk * h1
            else:
                ybuf[o + k, :] = dvk * ybuf[o + k, :]
        return carry

    lax.fori_loop(0, _RPT // 16, grp, 0)
    pltpu.sync_copy(ybuf, ytab_h.at[pl.ds(c * _NPAD + base, _RPT)])
    plsc.subcore_barrier()

    def gwait(j, m):
        pltpu.make_async_copy(ytab_h.at[sidx.at[j]], bufs.at[m],
                              gsem[m]).wait()

    def swait(m):
        pltpu.make_async_copy(bufs.at[m], acc.at[didx.at[0]],
                              ssem[m]).wait()

    pltpu.async_copy(ytab_h.at[sidx.at[0]], bufs.at[0], gsem[0])
    pltpu.async_copy(ytab_h.at[sidx.at[1]], bufs.at[1], gsem[1])

    for m in range(4):
        gwait(m, m)
        pltpu.async_copy(bufs.at[m], acc.at[didx.at[m]], ssem[m], add=True)
        m2 = (m + 2) % 4
        if m >= 2:
            swait(m2)
        pltpu.async_copy(ytab_h.at[sidx.at[m + 2]], bufs.at[m2], gsem[m2])

    def step(i, carry):
        for m in range(4):
            j = 4 * i + m
            gwait(j, m)
            pltpu.async_copy(bufs.at[m], acc.at[didx.at[j]], ssem[m],
                             add=True)
            m2 = (m + 2) % 4

            @pl.when(j + 2 < _CPT)
            def _():
                swait(m2)
                pltpu.async_copy(ytab_h.at[sidx.at[j + 2]], bufs.at[m2],
                                 gsem[m2])

        return carry

    lax.fori_loop(1, _CPT // 4, step, 0)
    for m in range(4):
        swait(m)
    plsc.subcore_barrier()
    pltpu.sync_copy(acc.at[pl.ds(base, _RPT)],
                    zout_h.at[c, pl.ds(base, _RPT)])


_mp_out = [jax.ShapeDtypeStruct((_NC, _NPAD, _H), jnp.float32),
           jax.ShapeDtypeStruct((_NC * _NPAD, _H), jnp.float32)]
_mp_scratch1 = [
    pltpu.VMEM((_CPT, _CH), jnp.int32),
    pltpu.VMEM((_CPT, _CH), jnp.int32),
    pltpu.VMEM((4, _CH, _H), jnp.float32),
    pltpu.VMEM((_NW, _RPT), jnp.float32),
    pltpu.VMEM((_RPT, _H), jnp.float32),
    pltpu.VMEM_SHARED((_NPAD, _H), jnp.float32),
] + [pltpu.SemaphoreType.DMA] * 8
_mp_scratch2 = _mp_scratch1[:5] + [
    pltpu.VMEM((2, _RPT, _H), jnp.float32),
    pltpu.VMEM((_H,), jnp.float32),
] + _mp_scratch1[5:]

_sc_mp1 = functools.partial(
    pl.kernel, out_type=_mp_out, mesh=_mesh, scratch_types=_mp_scratch1,
    compiler_params=_params)(functools.partial(_sc_mp_body, False))

_sc_mp2 = functools.partial(
    pl.kernel, out_type=_mp_out, mesh=_mesh, scratch_types=_mp_scratch2,
    compiler_params=_params)(functools.partial(_sc_mp_body, True))


def _tc_mm1(x_ref, w1_ref, xw_ref):
    xw_ref[...] = jnp.dot(x_ref[...], w1_ref[...],
                          preferred_element_type=jnp.float32)


def _tc_post(degp_ref, onesw_ref, xw1_ref, z1_ref, z2_ref, b1_ref, b2_ref,
             w2_ref, wfc_ref, bfc_ref, batch_ref, out_ref):
    deg = lax.dot_general(degp_ref[...], onesw_ref[...],
                          (((0,), (0,)), ((), ())),
                          preferred_element_type=jnp.float32) + 1.0
    dinv = lax.rsqrt(deg)
    y1 = dinv * xw1_ref[...]
    h1 = jnp.maximum(dinv * (z1_ref[0] + z1_ref[1] + y1) + b1_ref[...], 0.0)
    y2 = dinv * h1
    a = dinv * (z2_ref[0] + z2_ref[1] + y2)
    h2 = jnp.maximum(
        jnp.dot(a, w2_ref[...], preferred_element_type=jnp.float32)
        + b2_ref[...], 0.0)
    s = jnp.dot(h2, wfc_ref[...], preferred_element_type=jnp.float32)
    onehot = (lax.broadcasted_iota(jnp.int32, (_G, _NPAD), 0)
              == batch_ref[...]).astype(jnp.float32)
    pooled = jnp.dot(onehot, s, preferred_element_type=jnp.float32)
    logit = pooled + bfc_ref[...]
    out_ref[...] = 1.0 / (1.0 + jnp.exp(-logit))


def kernel(x, edge_index, batch, W1, b1, W2, b2, Wfc, bfc):
    f32 = jnp.float32
    src = edge_index[0].astype(jnp.int32)
    dst = edge_index[1].astype(jnp.int32)
    pad = _PADIDX + jnp.arange(_EPAD - _E, dtype=jnp.int32) % (_NPAD - _N)
    coff = (jnp.arange(_NW, dtype=jnp.int32) // _NS * _NPAD)[:, None, None]
    src_p = jnp.concatenate([src, pad]).reshape(_NW, _CPT, _CH) + coff
    dst_p = jnp.concatenate([dst, pad]).reshape(_NW, _CPT, _CH)
    x_p = jnp.pad(x, ((0, _NPAD - _N), (0, 0)))
    batch_p = jnp.pad(batch.astype(jnp.int32), (0, _NPAD - _N),
                      constant_values=_G).reshape(1, _NPAD)
    zeros16 = jnp.zeros((_NPAD, _H), f32)
    onesw = jnp.ones((_NW, 1), f32)
    b1r = b1.astype(f32)

    degp = _sc_deg(dst_p)

    xw1 = pl.pallas_call(
        _tc_mm1, out_shape=jax.ShapeDtypeStruct((_NPAD, _H), f32),
    )(x_p, W1)

    z1, _ = _sc_mp1(degp, xw1, src_p, dst_p, zeros16)
    z2, _ = _sc_mp2(degp, xw1, z1, b1r, src_p, dst_p, zeros16)

    out = pl.pallas_call(
        _tc_post, out_shape=jax.ShapeDtypeStruct((_G, 1), f32),
    )(degp, onesw, xw1, z1, z2, b1.reshape(1, _H), b2.reshape(1, _Y),
      W2, Wfc, bfc.reshape(1, 1), batch_p)

    return out
